# Initial kernel scaffold; baseline (speedup 1.0000x reference)
import functools

import jax
import jax.numpy as jnp
from jax import lax
from jax.experimental import pallas as pl
from jax.experimental.pallas import tpu as pltpu
from jax.experimental.pallas import tpu_sc as plsc

_LANES = 128
_NW = 32
_BLK = 1000


def _make_sc_agg(n_acc, width, nch):
    rpt = n_acc // 16
    mesh = plsc.VectorSubcoreMesh(core_axis_name="c", subcore_axis_name="s")

    @functools.partial(
        pl.kernel,
        out_type=jax.ShapeDtypeStruct((2, n_acc, width), jnp.float32),
        mesh=mesh,
        scratch_types=[
            pltpu.VMEM((nch, _LANES), jnp.int32),
            pltpu.VMEM((nch, _LANES), jnp.int32),
            pltpu.VMEM((_LANES, width), jnp.float32),
            pltpu.VMEM_SHARED((n_acc, width), jnp.float32),
            pltpu.SemaphoreType.DMA,
        ],
    )
    def sc_agg(table, src_idx, dst_idx, out, src_v, dst_v, rows_v, acc, sem):
        cid = lax.axis_index("c")
        sid = lax.axis_index("s")
        wid = sid * 2 + cid

        def _zero_row(i, carry):
            for c in range(width // 16):
                rows_v[i, pl.ds(c * 16, 16)] = jnp.zeros((16,), jnp.float32)
            return carry

        lax.fori_loop(0, _LANES, _zero_row, 0)
        base = sid * rpt
        for off in range(0, rpt, _LANES):
            pltpu.sync_copy(rows_v, acc.at[pl.ds(base + off, _LANES)])
        plsc.subcore_barrier()

        pltpu.sync_copy(src_idx.at[wid], src_v)
        pltpu.sync_copy(dst_idx.at[wid], dst_v)

        def _edge_chunk(j, carry):
            pltpu.async_copy(table.at[src_v.at[j]], rows_v, sem).wait()
            pltpu.sync_copy(rows_v, acc.at[dst_v.at[j]], add=True)
            return carry

        lax.fori_loop(0, nch, _edge_chunk, 0)
        plsc.subcore_barrier()

        for off in range(0, rpt, _LANES):
            pltpu.sync_copy(acc.at[pl.ds(base + off, _LANES)], rows_v)
            pltpu.sync_copy(rows_v, out.at[cid, pl.ds(base + off, _LANES)])

    return sc_agg


def _table1_body(x_ref, w_ref, o_ref):
    mm = jnp.dot(x_ref[...], w_ref[...], preferred_element_type=jnp.float32)
    col = lax.broadcasted_iota(jnp.int32, (mm.shape[0], 16), 1)
    extra = jnp.where(col == 0, 1.0, 0.0).astype(jnp.float32)
    o_ref[...] = jnp.concatenate([mm, extra], axis=1)


def _mid_body(d, x_ref, acc_ref, ws_ref, b_ref, wn2_ref, h1_ref, m2_ref, rd_ref):
    s = acc_ref[0] + acc_ref[1]
    agg = s[:, :d]
    deg = s[:, d : d + 1]
    rdeg = 1.0 / jnp.maximum(deg, 1.0)
    h = jnp.dot(x_ref[...], ws_ref[...], preferred_element_type=jnp.float32)
    h = jnp.maximum(h + agg * rdeg + b_ref[...], 0.0)
    h1_ref[...] = h
    m2_ref[...] = jnp.dot(h, wn2_ref[...], preferred_element_type=jnp.float32)
    rd_ref[...] = jnp.broadcast_to(rdeg, rd_ref.shape)


def _out_body(h1_ref, acc_ref, rd_ref, ws_ref, b_ref, o_ref):
    s = acc_ref[0] + acc_ref[1]
    o = jnp.dot(h1_ref[...], ws_ref[...], preferred_element_type=jnp.float32)
    o_ref[...] = o + s * rd_ref[...] + b_ref[...]


def kernel(x, edge_index, W_self1, W_neigh1, b1, W_self2, W_neigh2, b2):
    N, D = x.shape
    H = W_self1.shape[1]
    E = edge_index.shape[1]
    nch = -(-E // (_NW * _LANES))
    e_pad = nch * _NW * _LANES
    n_acc = -(-(N + 1) // 2048) * 2048
    w1 = H + 16
    grid = N // _BLK

    src = edge_index[0]
    dst = edge_index[1]
    pad = e_pad - E
    src_r = jnp.concatenate([src, jnp.zeros((pad,), jnp.int32)]).reshape(
        _NW, nch, _LANES)
    dst_r = jnp.concatenate([dst, jnp.full((pad,), N, jnp.int32)]).reshape(
        _NW, nch, _LANES)

    m1 = pl.pallas_call(
        _table1_body,
        grid=(grid,),
        in_specs=[
            pl.BlockSpec((_BLK, D), lambda i: (i, 0)),
            pl.BlockSpec((D, H), lambda i: (0, 0)),
        ],
        out_specs=pl.BlockSpec((_BLK, w1), lambda i: (i, 0)),
        out_shape=jax.ShapeDtypeStruct((N, w1), jnp.float32),
    )(x, W_neigh1)

    acc1 = _make_sc_agg(n_acc, w1, nch)(m1, src_r, dst_r)

    h1, m2, rdeg = pl.pallas_call(
        functools.partial(_mid_body, H),
        grid=(grid,),
        in_specs=[
            pl.BlockSpec((_BLK, D), lambda i: (i, 0)),
            pl.BlockSpec((2, _BLK, w1), lambda i: (0, i, 0)),
            pl.BlockSpec((D, H), lambda i: (0, 0)),
            pl.BlockSpec((1, H), lambda i: (0, 0)),
            pl.BlockSpec((H, H), lambda i: (0, 0)),
        ],
        out_specs=[
            pl.BlockSpec((_BLK, H), lambda i: (i, 0)),
            pl.BlockSpec((_BLK, H), lambda i: (i, 0)),
            pl.BlockSpec((_BLK, H), lambda i: (i, 0)),
        ],
        out_shape=[
            jax.ShapeDtypeStruct((N, H), jnp.float32),
            jax.ShapeDtypeStruct((N, H), jnp.float32),
            jax.ShapeDtypeStruct((N, H), jnp.float32),
        ],
    )(x, acc1, W_self1, b1.reshape(1, H), W_neigh2)

    acc2 = _make_sc_agg(n_acc, H, nch)(m2, src_r, dst_r)

    out = pl.pallas_call(
        _out_body,
        grid=(grid,),
        in_specs=[
            pl.BlockSpec((_BLK, H), lambda i: (i, 0)),
            pl.BlockSpec((2, _BLK, H), lambda i: (0, i, 0)),
            pl.BlockSpec((_BLK, H), lambda i: (i, 0)),
            pl.BlockSpec((H, H), lambda i: (0, 0)),
            pl.BlockSpec((1, H), lambda i: (0, 0)),
        ],
        out_specs=pl.BlockSpec((_BLK, H), lambda i: (i, 0)),
        out_shape=jax.ShapeDtypeStruct((N, H), jnp.float32),
    )(h1, acc2, rdeg, W_self2, b2.reshape(1, H))
    return out


# speedup vs baseline: 4.4182x; 4.4182x over previous
import functools

import jax
import jax.numpy as jnp
from jax import lax
from jax.experimental import pallas as pl
from jax.experimental.pallas import tpu as pltpu
from jax.experimental.pallas import tpu_sc as plsc

_LANES = 128
_NW = 32
_BLK = 1000


def _fill(ref, value, width):
    def _row(i, carry):
        for c in range(width // 16):
            ref[i, pl.ds(c * 16, 16)] = jnp.full((16,), value, jnp.float32)
        return carry

    lax.fori_loop(0, _LANES, _row, 0)


_MESH = plsc.VectorSubcoreMesh(
    core_axis_name="c", subcore_axis_name="s", num_cores=2, num_subcores=16)


def _make_sc_agg(n_acc, width, nch):
    rpt = n_acc // 16

    @functools.partial(
        pl.kernel,
        out_type=jax.ShapeDtypeStruct((2, n_acc, width), jnp.float32),
        mesh=_MESH,
        scratch_types=[
            pltpu.VMEM((nch, _LANES), jnp.int32),
            pltpu.VMEM((nch, _LANES), jnp.int32),
            pltpu.VMEM((_LANES, width), jnp.float32),
            pltpu.VMEM_SHARED((n_acc, width), jnp.float32),
            pltpu.SemaphoreType.DMA,
        ],
    )
    def sc_agg(table, src_idx, dst_idx, out, src_v, dst_v, rows_v, acc, sem):
        cid = lax.axis_index("c")
        sid = lax.axis_index("s")
        wid = sid * 2 + cid
        base = sid * rpt

        _fill(rows_v, 0.0, width)
        for off in range(0, rpt, _LANES):
            pltpu.sync_copy(rows_v, acc.at[pl.ds(base + off, _LANES)])
        plsc.subcore_barrier()

        pltpu.sync_copy(src_idx.at[wid], src_v)
        pltpu.sync_copy(dst_idx.at[wid], dst_v)

        def _edge_chunk(j, carry):
            pltpu.async_copy(table.at[src_v.at[j]], rows_v, sem).wait()
            pltpu.sync_copy(rows_v, acc.at[dst_v.at[j]], add=True)
            return carry

        lax.fori_loop(0, nch, _edge_chunk, 0)
        plsc.subcore_barrier()

        for off in range(0, rpt, _LANES):
            pltpu.sync_copy(acc.at[pl.ds(base + off, _LANES)], rows_v)
            pltpu.sync_copy(rows_v, out.at[cid, pl.ds(base + off, _LANES)])

    return sc_agg


def _make_sc_deg(n_acc, nch, dw=128):
    rpt = n_acc // 16

    @functools.partial(
        pl.kernel,
        out_type=jax.ShapeDtypeStruct((2, n_acc, dw), jnp.float32),
        mesh=_MESH,
        scratch_types=[
            pltpu.VMEM((nch, _LANES), jnp.int32),
            pltpu.VMEM((_LANES, dw), jnp.float32),
            pltpu.VMEM_SHARED((n_acc, dw), jnp.float32),
        ],
    )
    def sc_deg(dst_idx, out, dst_v, ones_v, dacc):
        cid = lax.axis_index("c")
        sid = lax.axis_index("s")
        wid = sid * 2 + cid
        base = sid * rpt

        _fill(ones_v, 0.0, dw)
        for off in range(0, rpt, _LANES):
            pltpu.sync_copy(ones_v, dacc.at[pl.ds(base + off, _LANES)])
        _fill(ones_v, 1.0, dw)
        plsc.subcore_barrier()

        pltpu.sync_copy(dst_idx.at[wid], dst_v)

        def _edge_chunk(j, carry):
            pltpu.sync_copy(ones_v, dacc.at[dst_v.at[j]], add=True)
            return carry

        lax.fori_loop(0, nch, _edge_chunk, 0)
        plsc.subcore_barrier()

        for off in range(0, rpt, _LANES):
            pltpu.sync_copy(dacc.at[pl.ds(base + off, _LANES)], ones_v)
            pltpu.sync_copy(ones_v, out.at[cid, pl.ds(base + off, _LANES)])

    return sc_deg


def _table1_body(x_ref, w_ref, o_ref):
    o_ref[...] = jnp.dot(
        x_ref[...], w_ref[...], preferred_element_type=jnp.float32)


def _mid_body(x_ref, acc_ref, deg_ref, ws_ref, b_ref, wn2_ref,
              h1_ref, m2_ref, rd_ref):
    agg = acc_ref[0] + acc_ref[1]
    deg = (deg_ref[0] + deg_ref[1])[:, 0:1]
    rdeg = 1.0 / jnp.maximum(deg, 1.0)
    h = jnp.dot(x_ref[...], ws_ref[...], preferred_element_type=jnp.float32)
    h = jnp.maximum(h + agg * rdeg + b_ref[...], 0.0)
    h1_ref[...] = h
    m2_ref[...] = jnp.dot(h, wn2_ref[...], preferred_element_type=jnp.float32)
    rd_ref[...] = jnp.broadcast_to(rdeg, rd_ref.shape)


def _out_body(h1_ref, acc_ref, rd_ref, ws_ref, b_ref, o_ref):
    s = acc_ref[0] + acc_ref[1]
    o = jnp.dot(h1_ref[...], ws_ref[...], preferred_element_type=jnp.float32)
    o_ref[...] = o + s * rd_ref[...] + b_ref[...]


def kernel(x, edge_index, W_self1, W_neigh1, b1, W_self2, W_neigh2, b2):
    N, D = x.shape
    H = W_self1.shape[1]
    E = edge_index.shape[1]
    nch = -(-E // (_NW * _LANES))
    e_pad = nch * _NW * _LANES
    n_acc = -(-(N + 1) // 2048) * 2048
    grid = N // _BLK

    src = edge_index[0]
    dst = edge_index[1]
    pad = e_pad - E
    src_r = jnp.concatenate([src, jnp.zeros((pad,), jnp.int32)]).reshape(
        _NW, nch, _LANES)
    dst_r = jnp.concatenate([dst, jnp.full((pad,), N, jnp.int32)]).reshape(
        _NW, nch, _LANES)

    m1 = pl.pallas_call(
        _table1_body,
        grid=(grid,),
        in_specs=[
            pl.BlockSpec((_BLK, D), lambda i: (i, 0)),
            pl.BlockSpec((D, H), lambda i: (0, 0)),
        ],
        out_specs=pl.BlockSpec((_BLK, H), lambda i: (i, 0)),
        out_shape=jax.ShapeDtypeStruct((N, H), jnp.float32),
    )(x, W_neigh1)

    deg1 = _make_sc_deg(n_acc, nch)(dst_r)
    acc1 = _make_sc_agg(n_acc, H, nch)(m1, src_r, dst_r)

    h1, m2, rdeg = pl.pallas_call(
        _mid_body,
        grid=(grid,),
        in_specs=[
            pl.BlockSpec((_BLK, D), lambda i: (i, 0)),
            pl.BlockSpec((2, _BLK, H), lambda i: (0, i, 0)),
            pl.BlockSpec((2, _BLK, 128), lambda i: (0, i, 0)),
            pl.BlockSpec((D, H), lambda i: (0, 0)),
            pl.BlockSpec((1, H), lambda i: (0, 0)),
            pl.BlockSpec((H, H), lambda i: (0, 0)),
        ],
        out_specs=[
            pl.BlockSpec((_BLK, H), lambda i: (i, 0)),
            pl.BlockSpec((_BLK, H), lambda i: (i, 0)),
            pl.BlockSpec((_BLK, H), lambda i: (i, 0)),
        ],
        out_shape=[
            jax.ShapeDtypeStruct((N, H), jnp.float32),
            jax.ShapeDtypeStruct((N, H), jnp.float32),
            jax.ShapeDtypeStruct((N, H), jnp.float32),
        ],
    )(x, acc1, deg1, W_self1, b1.reshape(1, H), W_neigh2)

    acc2 = _make_sc_agg(n_acc, H, nch)(m2, src_r, dst_r)

    out = pl.pallas_call(
        _out_body,
        grid=(grid,),
        in_specs=[
            pl.BlockSpec((_BLK, H), lambda i: (i, 0)),
            pl.BlockSpec((2, _BLK, H), lambda i: (0, i, 0)),
            pl.BlockSpec((_BLK, H), lambda i: (i, 0)),
            pl.BlockSpec((H, H), lambda i: (0, 0)),
            pl.BlockSpec((1, H), lambda i: (0, 0)),
        ],
        out_specs=pl.BlockSpec((_BLK, H), lambda i: (i, 0)),
        out_shape=jax.ShapeDtypeStruct((N, H), jnp.float32),
    )(h1, acc2, rdeg, W_self2, b2.reshape(1, H))
    return out
